# baseline (speedup 1.0000x reference)
import functools

import jax
import jax.numpy as jnp
from jax import lax
from jax.experimental import pallas as pl
from jax.experimental.pallas import tpu as pltpu
from jax.experimental.pallas import tpu_sc as plsc

VOCAB_SIZE = 100000
D_DIM = 16
B_DIM = 1024

_INFO = plsc.get_sparse_core_info()
_NC = _INFO.num_cores
_NW = _INFO.num_cores * _INFO.num_subcores
_BPW = B_DIM // _NW

_MESH = plsc.VectorSubcoreMesh(core_axis_name="c", subcore_axis_name="s")


@functools.partial(
    pl.kernel,
    mesh=_MESH,
    out_type=jax.ShapeDtypeStruct((B_DIM, D_DIM), jnp.float32),
    scratch_types=[
        pltpu.VMEM((_BPW,), jnp.int32),
        pltpu.VMEM((_BPW, D_DIM), jnp.float32),
        pltpu.SemaphoreType.DMA,
    ],
    compiler_params=pltpu.CompilerParams(use_tc_tiling_on_sc=False),
)
def _sc_gather(idx_hbm, table_hbm, out_hbm, idx_v, rows_v, sem):
    wid = lax.axis_index("s") * _NC + lax.axis_index("c")
    base = wid * _BPW
    pltpu.sync_copy(idx_hbm.at[pl.ds(base, _BPW)], idx_v)
    pltpu.async_copy(table_hbm.at[idx_v], rows_v, sem).wait()
    pltpu.sync_copy(rows_v, out_hbm.at[pl.ds(base, _BPW)])


_VB = 2048
_NBLK = (VOCAB_SIZE + _VB - 1) // _VB


def _mm_body(emb_ref, w_ref, out_ref):
    out_ref[...] = lax.dot_general(
        emb_ref[...],
        w_ref[...],
        dimension_numbers=(((1,), (1,)), ((), ())),
        preferred_element_type=jnp.float32,
    )


def _project(emb, W):
    return pl.pallas_call(
        _mm_body,
        grid=(_NBLK,),
        in_specs=[
            pl.BlockSpec((B_DIM, D_DIM), lambda i: (0, 0)),
            pl.BlockSpec((_VB, D_DIM), lambda i: (i, 0)),
        ],
        out_specs=pl.BlockSpec((B_DIM, _VB), lambda i: (0, i)),
        out_shape=jax.ShapeDtypeStruct((B_DIM, VOCAB_SIZE), jnp.float32),
    )(emb, W)


def kernel(indices, table, W):
    emb = _sc_gather(indices.astype(jnp.int32), table)
    return _project(emb, W)


# speedup vs baseline: 1.0759x; 1.0759x over previous
import functools

import jax
import jax.numpy as jnp
from jax import lax
from jax.experimental import pallas as pl
from jax.experimental.pallas import tpu as pltpu
from jax.experimental.pallas import tpu_sc as plsc

VOCAB_SIZE = 100000
D_DIM = 16
B_DIM = 1024

_INFO = plsc.get_sparse_core_info()
_NC = _INFO.num_cores
_NW = _INFO.num_cores * _INFO.num_subcores
_BPW = B_DIM // _NW

_MESH = plsc.VectorSubcoreMesh(core_axis_name="c", subcore_axis_name="s")


@functools.partial(
    pl.kernel,
    mesh=_MESH,
    out_type=jax.ShapeDtypeStruct((B_DIM, D_DIM), jnp.float32),
    scratch_types=[
        pltpu.VMEM((_BPW,), jnp.int32),
        pltpu.VMEM((_BPW, D_DIM), jnp.float32),
        pltpu.SemaphoreType.DMA,
    ],
    compiler_params=pltpu.CompilerParams(use_tc_tiling_on_sc=False),
)
def _sc_gather(idx_hbm, table_hbm, out_hbm, idx_v, rows_v, sem):
    wid = lax.axis_index("s") * _NC + lax.axis_index("c")
    base = wid * _BPW
    pltpu.sync_copy(idx_hbm.at[pl.ds(base, _BPW)], idx_v)
    pltpu.async_copy(table_hbm.at[idx_v], rows_v, sem).wait()
    pltpu.sync_copy(rows_v, out_hbm.at[pl.ds(base, _BPW)])


_RB = 16
_NSTEP = B_DIM // _RB
_K = 6


def _mm_body(emb_ref, wt_ref, out_ref, ring, sems):
    i = pl.program_id(0)
    s = lax.rem(i, _K)

    @pl.when(i >= _K)
    def _wait_prev():
        pltpu.make_async_copy(
            ring.at[s],
            out_ref.at[pl.ds((i - _K) * _RB, _RB), :],
            sems.at[s],
        ).wait()

    ring[s] = lax.dot_general(
        emb_ref[...],
        wt_ref[...],
        dimension_numbers=(((1,), (0,)), ((), ())),
        preferred_element_type=jnp.float32,
    )
    pltpu.make_async_copy(
        ring.at[s],
        out_ref.at[pl.ds(i * _RB, _RB), :],
        sems.at[s],
    ).start()

    @pl.when(i == _NSTEP - 1)
    def _drain():
        for j in range(_K):
            t = _NSTEP - _K + j
            pltpu.make_async_copy(
                ring.at[t % _K],
                out_ref.at[pl.ds(t * _RB, _RB), :],
                sems.at[t % _K],
            ).wait()


def _project(emb, Wt):
    return pl.pallas_call(
        _mm_body,
        grid=(_NSTEP,),
        in_specs=[
            pl.BlockSpec((_RB, D_DIM), lambda i: (i, 0)),
            pl.BlockSpec((D_DIM, VOCAB_SIZE), lambda i: (0, 0)),
        ],
        out_specs=pl.BlockSpec(memory_space=pltpu.MemorySpace.HBM),
        out_shape=jax.ShapeDtypeStruct((B_DIM, VOCAB_SIZE), jnp.float32),
        scratch_shapes=[
            pltpu.VMEM((_K, _RB, VOCAB_SIZE), jnp.float32),
            pltpu.SemaphoreType.DMA((_K,)),
        ],
    )(emb, Wt)


def kernel(indices, table, W):
    emb = _sc_gather(indices.astype(jnp.int32), table)
    return _project(emb, W.T)


# speedup vs baseline: 1.0767x; 1.0008x over previous
import functools

import jax
import jax.numpy as jnp
from jax import lax
from jax.experimental import pallas as pl
from jax.experimental.pallas import tpu as pltpu
from jax.experimental.pallas import tpu_sc as plsc

VOCAB_SIZE = 100000
D_DIM = 16
B_DIM = 1024

_INFO = plsc.get_sparse_core_info()
_NC = _INFO.num_cores
_NW = _INFO.num_cores * _INFO.num_subcores
_BPW = B_DIM // _NW

_MESH = plsc.VectorSubcoreMesh(core_axis_name="c", subcore_axis_name="s")


@functools.partial(
    pl.kernel,
    mesh=_MESH,
    out_type=jax.ShapeDtypeStruct((B_DIM, D_DIM), jnp.float32),
    scratch_types=[
        pltpu.VMEM((_BPW,), jnp.int32),
        pltpu.VMEM((_BPW, D_DIM), jnp.float32),
        pltpu.SemaphoreType.DMA,
    ],
    compiler_params=pltpu.CompilerParams(use_tc_tiling_on_sc=False),
)
def _sc_gather(idx_hbm, table_hbm, out_hbm, idx_v, rows_v, sem):
    wid = lax.axis_index("s") * _NC + lax.axis_index("c")
    base = wid * _BPW
    pltpu.sync_copy(idx_hbm.at[pl.ds(base, _BPW)], idx_v)
    pltpu.async_copy(table_hbm.at[idx_v], rows_v, sem).wait()
    pltpu.sync_copy(rows_v, out_hbm.at[pl.ds(base, _BPW)])


_RB = 64
_NSTEP = B_DIM // _RB
_K = 2


def _mm_body(emb_ref, wt_ref, out_ref, ring, sems):
    i = pl.program_id(0)
    s = lax.rem(i, _K)

    @pl.when(i >= _K)
    def _wait_prev():
        pltpu.make_async_copy(
            ring.at[s],
            out_ref.at[pl.ds((i - _K) * _RB, _RB), :],
            sems.at[s],
        ).wait()

    ring[s] = lax.dot_general(
        emb_ref[...],
        wt_ref[...],
        dimension_numbers=(((1,), (0,)), ((), ())),
        preferred_element_type=jnp.float32,
    )
    pltpu.make_async_copy(
        ring.at[s],
        out_ref.at[pl.ds(i * _RB, _RB), :],
        sems.at[s],
    ).start()

    @pl.when(i == _NSTEP - 1)
    def _drain():
        for j in range(_K):
            t = _NSTEP - _K + j
            pltpu.make_async_copy(
                ring.at[t % _K],
                out_ref.at[pl.ds(t * _RB, _RB), :],
                sems.at[t % _K],
            ).wait()


def _project(emb, Wt):
    return pl.pallas_call(
        _mm_body,
        grid=(_NSTEP,),
        in_specs=[
            pl.BlockSpec((_RB, D_DIM), lambda i: (i, 0)),
            pl.BlockSpec((D_DIM, VOCAB_SIZE), lambda i: (0, 0)),
        ],
        out_specs=pl.BlockSpec(memory_space=pltpu.MemorySpace.HBM),
        out_shape=jax.ShapeDtypeStruct((B_DIM, VOCAB_SIZE), jnp.float32),
        scratch_shapes=[
            pltpu.VMEM((_K, _RB, VOCAB_SIZE), jnp.float32),
            pltpu.SemaphoreType.DMA((_K,)),
        ],
        compiler_params=pltpu.CompilerParams(
            vmem_limit_bytes=110 * 1024 * 1024,
        ),
    )(emb, Wt)


def kernel(indices, table, W):
    emb = _sc_gather(indices.astype(jnp.int32), table)
    return _project(emb, W.T)


# speedup vs baseline: 2.5286x; 2.3484x over previous
import functools

import jax
import jax.numpy as jnp
from jax import lax
from jax.experimental import pallas as pl
from jax.experimental.pallas import tpu as pltpu
from jax.experimental.pallas import tpu_sc as plsc

VOCAB_SIZE = 100000
D_DIM = 16
B_DIM = 1024

_INFO = plsc.get_sparse_core_info()
_NC = _INFO.num_cores
_NW = _INFO.num_cores * _INFO.num_subcores
_BPW = B_DIM // _NW

_MESH = plsc.VectorSubcoreMesh(core_axis_name="c", subcore_axis_name="s")


@functools.partial(
    pl.kernel,
    mesh=_MESH,
    out_type=jax.ShapeDtypeStruct((B_DIM, D_DIM), jnp.float32),
    scratch_types=[
        pltpu.VMEM((_BPW,), jnp.int32),
        pltpu.VMEM((_BPW, D_DIM), jnp.float32),
        pltpu.SemaphoreType.DMA,
    ],
    compiler_params=pltpu.CompilerParams(use_tc_tiling_on_sc=False),
)
def _sc_gather(idx_hbm, table_hbm, out_hbm, idx_v, rows_v, sem):
    wid = lax.axis_index("s") * _NC + lax.axis_index("c")
    base = wid * _BPW
    pltpu.sync_copy(idx_hbm.at[pl.ds(base, _BPW)], idx_v)
    pltpu.async_copy(table_hbm.at[idx_v], rows_v, sem).wait()
    pltpu.sync_copy(rows_v, out_hbm.at[pl.ds(base, _BPW)])


_VB = 2048
_NBLK = (VOCAB_SIZE + _VB - 1) // _VB
_TAIL = VOCAB_SIZE - (_NBLK - 1) * _VB
_K = 4


def _mm_body(w_ref, emb_ref, out_ref, ring, sems):
    i = pl.program_id(0)
    s = lax.rem(i, _K)

    @pl.when(i >= _K)
    def _wait_prev():
        pltpu.make_async_copy(
            ring.at[s],
            out_ref.at[pl.ds((i - _K) * _VB, _VB), :],
            sems.at[s],
        ).wait()

    ring[s] = lax.dot_general(
        w_ref[...],
        emb_ref[...],
        dimension_numbers=(((1,), (1,)), ((), ())),
        preferred_element_type=jnp.float32,
    )

    @pl.when(i < _NBLK - 1)
    def _start_full():
        pltpu.make_async_copy(
            ring.at[s],
            out_ref.at[pl.ds(i * _VB, _VB), :],
            sems.at[s],
        ).start()

    @pl.when(i == _NBLK - 1)
    def _start_tail_and_drain():
        pltpu.make_async_copy(
            ring.at[s, : _TAIL, :],
            out_ref.at[pl.ds((_NBLK - 1) * _VB, _TAIL), :],
            sems.at[s],
        ).start()
        for j in range(_K):
            t = _NBLK - _K + j
            if t == _NBLK - 1:
                pltpu.make_async_copy(
                    ring.at[t % _K, : _TAIL, :],
                    out_ref.at[pl.ds(t * _VB, _TAIL), :],
                    sems.at[t % _K],
                ).wait()
            else:
                pltpu.make_async_copy(
                    ring.at[t % _K],
                    out_ref.at[pl.ds(t * _VB, _VB), :],
                    sems.at[t % _K],
                ).wait()


def _project_t(W, emb):
    return pl.pallas_call(
        _mm_body,
        grid=(_NBLK,),
        in_specs=[
            pl.BlockSpec((_VB, D_DIM), lambda i: (i, 0)),
            pl.BlockSpec((B_DIM, D_DIM), lambda i: (0, 0)),
        ],
        out_specs=pl.BlockSpec(memory_space=pltpu.MemorySpace.HBM),
        out_shape=jax.ShapeDtypeStruct((VOCAB_SIZE, B_DIM), jnp.float32),
        scratch_shapes=[
            pltpu.VMEM((_K, _VB, B_DIM), jnp.float32),
            pltpu.SemaphoreType.DMA((_K,)),
        ],
        compiler_params=pltpu.CompilerParams(
            vmem_limit_bytes=100 * 1024 * 1024,
        ),
    )(W, emb)


def kernel(indices, table, W):
    emb = _sc_gather(indices.astype(jnp.int32), table)
    return _project_t(W, emb).T


# speedup vs baseline: 3.0582x; 1.2095x over previous
import functools

import jax
import jax.numpy as jnp
from jax import lax
from jax.experimental import pallas as pl
from jax.experimental.pallas import tpu as pltpu
from jax.experimental.pallas import tpu_sc as plsc

VOCAB_SIZE = 100000
D_DIM = 16
B_DIM = 1024

_INFO = plsc.get_sparse_core_info()
_NC = _INFO.num_cores
_NW = _INFO.num_cores * _INFO.num_subcores
_BPW = B_DIM // _NW

_MESH = plsc.VectorSubcoreMesh(core_axis_name="c", subcore_axis_name="s")


@functools.partial(
    pl.kernel,
    mesh=_MESH,
    out_type=jax.ShapeDtypeStruct((B_DIM, D_DIM), jnp.float32),
    scratch_types=[
        pltpu.VMEM((_BPW,), jnp.int32),
        pltpu.VMEM((_BPW, D_DIM), jnp.float32),
        pltpu.SemaphoreType.DMA,
    ],
    compiler_params=pltpu.CompilerParams(use_tc_tiling_on_sc=False),
)
def _sc_gather(idx_hbm, table_hbm, out_hbm, idx_v, rows_v, sem):
    wid = lax.axis_index("s") * _NC + lax.axis_index("c")
    base = wid * _BPW
    pltpu.sync_copy(idx_hbm.at[pl.ds(base, _BPW)], idx_v)
    pltpu.async_copy(table_hbm.at[idx_v], rows_v, sem).wait()
    pltpu.sync_copy(rows_v, out_hbm.at[pl.ds(base, _BPW)])


_VB = 2048
_NBLK = (VOCAB_SIZE + _VB - 1) // _VB
_TAIL = VOCAB_SIZE - (_NBLK - 1) * _VB
_K = 4


def _mm_body(wt_ref, emb_ref, out_ref, ring, sems):
    i = pl.program_id(0)
    s = lax.rem(i, _K)

    @pl.when(i >= _K)
    def _wait_prev():
        pltpu.make_async_copy(
            ring.at[s],
            out_ref.at[pl.ds((i - _K) * _VB, _VB), :],
            sems.at[s],
        ).wait()

    ring[s] = lax.dot_general(
        wt_ref[...],
        emb_ref[...],
        dimension_numbers=(((0,), (1,)), ((), ())),
        preferred_element_type=jnp.float32,
    )

    @pl.when(i < _NBLK - 1)
    def _start_full():
        pltpu.make_async_copy(
            ring.at[s],
            out_ref.at[pl.ds(i * _VB, _VB), :],
            sems.at[s],
        ).start()

    @pl.when(i == _NBLK - 1)
    def _start_tail_and_drain():
        pltpu.make_async_copy(
            ring.at[s, : _TAIL, :],
            out_ref.at[pl.ds((_NBLK - 1) * _VB, _TAIL), :],
            sems.at[s],
        ).start()
        for j in range(_K):
            t = _NBLK - _K + j
            if t == _NBLK - 1:
                pltpu.make_async_copy(
                    ring.at[t % _K, : _TAIL, :],
                    out_ref.at[pl.ds(t * _VB, _TAIL), :],
                    sems.at[t % _K],
                ).wait()
            else:
                pltpu.make_async_copy(
                    ring.at[t % _K],
                    out_ref.at[pl.ds(t * _VB, _VB), :],
                    sems.at[t % _K],
                ).wait()


def _project_t(Wt, emb):
    return pl.pallas_call(
        _mm_body,
        grid=(_NBLK,),
        in_specs=[
            pl.BlockSpec((D_DIM, _VB), lambda i: (0, i)),
            pl.BlockSpec((B_DIM, D_DIM), lambda i: (0, 0)),
        ],
        out_specs=pl.BlockSpec(memory_space=pltpu.MemorySpace.HBM),
        out_shape=jax.ShapeDtypeStruct((VOCAB_SIZE, B_DIM), jnp.float32),
        scratch_shapes=[
            pltpu.VMEM((_K, _VB, B_DIM), jnp.float32),
            pltpu.SemaphoreType.DMA((_K,)),
        ],
        compiler_params=pltpu.CompilerParams(
            vmem_limit_bytes=100 * 1024 * 1024,
        ),
    )(Wt, emb)


def kernel(indices, table, W):
    emb = _sc_gather(indices.astype(jnp.int32), table)
    return _project_t(W.T, emb).T
